# baseline (speedup 1.0000x reference)
import functools

import jax
import jax.numpy as jnp
from jax import lax
from jax.experimental import pallas as pl
from jax.experimental.pallas import tpu as pltpu
from jax.experimental.pallas import tpu_sc as plsc

N = 10000
NP = 10240
E = 160000
EP = 163840
H = 3
HID = 256
D = 768
NW = 32
TPW = EP // NW
NB = 5
CB = TPW // NB
C = 32
KPB = CB // C
NT = H * NP
SLICE = NT // 16
PARTS = 4
PLEN = NT // PARTS
PSL = PLEN // 16


def _prep_body(pe_ref, wl_ref, wr_ref, bl_ref, br_ref, w3t_ref, xl_out, xr_out, pt_out):
    hi = jax.lax.Precision.HIGHEST
    xl = jnp.dot(pe_ref[...], wl_ref[...], preferred_element_type=jnp.float32,
                 precision=hi) + bl_ref[...]
    xl_out[...] = xl.astype(jnp.bfloat16)
    xr = jnp.dot(pe_ref[...], wr_ref[...], preferred_element_type=jnp.float32,
                 precision=hi) + br_ref[...]
    xr_out[...] = xr.astype(jnp.bfloat16)
    pt_out[...] = jax.lax.dot_general(
        w3t_ref[...], xl, (((1,), (1,)), ((), ())),
        preferred_element_type=jnp.float32, precision=hi)


def _prep(pe_p, Wl, Wr, bl2, br2, w3t):
    bm = 256
    grid = (NP // bm,)
    return pl.pallas_call(
        _prep_body,
        grid=grid,
        in_specs=[
            pl.BlockSpec((bm, 256), lambda i: (i, 0)),
            pl.BlockSpec((256, D), lambda i: (0, 0)),
            pl.BlockSpec((256, D), lambda i: (0, 0)),
            pl.BlockSpec((1, D), lambda i: (0, 0)),
            pl.BlockSpec((1, D), lambda i: (0, 0)),
            pl.BlockSpec((H, D), lambda i: (0, 0)),
        ],
        out_specs=[
            pl.BlockSpec((bm, D), lambda i: (i, 0)),
            pl.BlockSpec((bm, D), lambda i: (i, 0)),
            pl.BlockSpec((H, bm), lambda i: (0, i)),
        ],
        out_shape=[
            jax.ShapeDtypeStruct((NP, D), jnp.bfloat16),
            jax.ShapeDtypeStruct((NP, D), jnp.bfloat16),
            jax.ShapeDtypeStruct((H, NP), jnp.float32),
        ],
    )(pe_p, Wl, Wr, bl2, br2, w3t)


def _worker_id():
    return lax.axis_index("s") * 2 + lax.axis_index("c")


def _tree_reduce(loc, tmpa, tmpb0, tmpb1, shared_all, out_ref, cid, sid, semr):
    sl0 = sid * PSL

    def fetch(t, buf):
        pltpu.async_copy(shared_all.at[t, pl.ds(sl0, PSL)], buf, semr)

    def wait(buf):
        pltpu.make_async_copy(shared_all.at[0, pl.ds(sl0, PSL)], buf, semr).wait()

    def addin(buf):
        def add_body(k, _):
            s = pl.ds(k * 16, 16)
            tmpa[s] = tmpa[s] + buf[s]
            return 0

        lax.fori_loop(0, PSL // 16, add_body, 0, unroll=8)

    def part_body(part, _):
        pltpu.sync_copy(loc.at[pl.ds(part * PLEN, PLEN)], shared_all.at[sid])
        plsc.subcore_barrier()
        pltpu.sync_copy(shared_all.at[0, pl.ds(sl0, PSL)], tmpa)
        fetch(1, tmpb0)
        for pi in range(7):
            wait(tmpb0)
            fetch(2 * pi + 2, tmpb1)
            addin(tmpb0)
            wait(tmpb1)
            fetch(2 * pi + 3, tmpb0)
            addin(tmpb1)
        wait(tmpb0)
        addin(tmpb0)
        pltpu.sync_copy(tmpa, out_ref.at[pl.ds(cid * NT + part * PLEN + sl0, PSL)])
        plsc.subcore_barrier()
        return 0

    lax.fori_loop(0, PARTS, part_body, 0)


def _pass1_body(src_r, dst_r, f_r, xl_r, xr_r, we_r, att_r,
                ex_r, den_r,
                sbuf_src, sbuf_dst, sbuf_f, rows_la, rows_ra, rows_lb, rows_rb,
                webuf, attbuf, exbig, den_loc, tmpa, tmpb0, tmpb1, shared_all,
                sem_la, sem_ra, sem_lb, sem_rb):
    cid = lax.axis_index("c")
    sid = lax.axis_index("s")
    wid = _worker_id()
    base_row = wid * (TPW // C)

    pltpu.sync_copy(we_r, webuf)
    pltpu.sync_copy(att_r, attbuf)

    zv = jnp.zeros((16,), jnp.float32)

    def zero_body(k, _):
        den_loc[pl.ds(k * 16, 16)] = zv
        return 0

    lax.fori_loop(0, NT // 16, zero_body, 0, unroll=8)

    lane16 = lax.iota(jnp.int32, 16)
    perms = [(lane16 + s) & 15 for s in (8, 4, 2, 1)]

    def start(k, buf_l, buf_r, sl, sr):
        pltpu.async_copy(xl_r.at[sbuf_src.at[k]], buf_l, sl)
        pltpu.async_copy(xr_r.at[sbuf_dst.at[k]], buf_r, sr)

    def drain(buf_l, buf_r, sl, sr):
        pltpu.make_async_copy(xl_r.at[pl.ds(0, C)], buf_l, sl).wait()
        pltpu.make_async_copy(xr_r.at[pl.ds(0, C)], buf_r, sr).wait()

    p2v = jnp.full((32,), 0.2, jnp.bfloat16)

    def compute(k, rows_l, rows_r):
        for j in range(C // 16):
            fv = sbuf_f[k, pl.ds(j * 16, 16)]
            dstv = sbuf_dst[k, pl.ds(j * 16, 16)]
            for h in range(H):
                we_regs = [webuf[pl.ds(h * HID + c * 32, 32)] for c in range(8)]
                att_regs = [attbuf[pl.ds(h * HID + c * 32, 32)] for c in range(8)]

                def e_body(e, alpha_vec):
                    esp = jnp.full((16,), e, jnp.int32)
                    fsp = fv.at[esp].get(mode="promise_in_bounds")
                    fsb = plsc.pack(fsp, fsp, format=plsc.PackFormat.INTERLEAVED)
                    acc_bf = jnp.zeros((32,), jnp.bfloat16)
                    for c in range(8):
                        s = pl.ds(h * HID + c * 32, 32)
                        y = (rows_l[j * 16 + e, s] + rows_r[j * 16 + e, s]
                             + fsb * we_regs[c])
                        y = jnp.maximum(y, p2v * y)
                        acc_bf = acc_bf + att_regs[c] * y
                    lo, hi = plsc.unpack(acc_bf, format=plsc.PackFormat.INTERLEAVED)
                    acc = lo + hi
                    for p in perms:
                        acc = acc + acc.at[p].get(mode="promise_in_bounds")
                    return jnp.where(lane16 == e, acc, alpha_vec)

                alpha = lax.fori_loop(0, 16, e_body, jnp.zeros((16,), jnp.float32),
                                      unroll=2)
                exv = jnp.exp(alpha)
                exbig[h, pl.ds((k % KPB) * C + j * 16, 16)] = exv
                plsc.addupdate_scatter(den_loc, [dstv + h * NP], exv)

    def batch_body(b, _):
        row0 = base_row + b * KPB
        pltpu.sync_copy(src_r.at[pl.ds(row0, KPB)], sbuf_src)
        pltpu.sync_copy(dst_r.at[pl.ds(row0, KPB)], sbuf_dst)
        pltpu.sync_copy(f_r.at[pl.ds(row0, KPB)], sbuf_f)

        start(0, rows_la, rows_ra, sem_la, sem_ra)

        def pair_body(p, _):
            k0 = 2 * p
            drain(rows_la, rows_ra, sem_la, sem_ra)
            start(k0 + 1, rows_lb, rows_rb, sem_lb, sem_rb)
            compute(k0, rows_la, rows_ra)
            drain(rows_lb, rows_rb, sem_lb, sem_rb)
            start(jnp.minimum(k0 + 2, KPB - 1), rows_la, rows_ra, sem_la, sem_ra)
            compute(k0 + 1, rows_lb, rows_rb)
            return 0

        lax.fori_loop(0, KPB // 2, pair_body, 0)
        drain(rows_la, rows_ra, sem_la, sem_ra)
        for h in range(H):
            pltpu.sync_copy(exbig.at[h], ex_r.at[pl.ds(h * EP + row0 * C, CB)])
        return 0

    lax.fori_loop(0, NB, batch_body, 0)

    plsc.subcore_barrier()
    _tree_reduce(den_loc, tmpa, tmpb0, tmpb1, shared_all, den_r, cid, sid, sem_la)


def _pass1(src2, dst2, f2, XL, XR, wef, attf):
    k = pl.kernel(
        _pass1_body,
        out_type=[
            jax.ShapeDtypeStruct((H * EP,), jnp.float32),
            jax.ShapeDtypeStruct((2 * NT,), jnp.float32),
        ],
        mesh=plsc.VectorSubcoreMesh(core_axis_name="c", subcore_axis_name="s"),
        compiler_params=pltpu.CompilerParams(use_tc_tiling_on_sc=False, needs_layout_passes=False),
        scratch_types=[
            pltpu.VMEM((KPB, C), jnp.int32),
            pltpu.VMEM((KPB, C), jnp.int32),
            pltpu.VMEM((KPB, C), jnp.float32),
            pltpu.VMEM((C, D), jnp.bfloat16),
            pltpu.VMEM((C, D), jnp.bfloat16),
            pltpu.VMEM((C, D), jnp.bfloat16),
            pltpu.VMEM((C, D), jnp.bfloat16),
            pltpu.VMEM((D,), jnp.bfloat16),
            pltpu.VMEM((D,), jnp.bfloat16),
            pltpu.VMEM((H, CB), jnp.float32),
            pltpu.VMEM((NT,), jnp.float32),
            pltpu.VMEM((PSL,), jnp.float32),
            pltpu.VMEM((PSL,), jnp.float32),
            pltpu.VMEM((PSL,), jnp.float32),
            pltpu.VMEM_SHARED((16, PLEN), jnp.float32),
            pltpu.SemaphoreType.DMA,
            pltpu.SemaphoreType.DMA,
            pltpu.SemaphoreType.DMA,
            pltpu.SemaphoreType.DMA,
        ],
    )
    return k(src2, dst2, f2, XL, XR, wef, attf)


def _pass2_body(src_r, dst_r, ex_r, den_r, p_r,
                q_r,
                sbuf_src, sbuf_dst, exb, p_loc, inv_loc, q_loc,
                tmpa, tmpb0, tmpb1, shared_all, semr):
    cid = lax.axis_index("c")
    sid = lax.axis_index("s")
    wid = _worker_id()
    base_row = wid * (TPW // C)

    for h in range(H):
        pltpu.sync_copy(p_r.at[h], p_loc.at[pl.ds(h * NP, NP)])
    pltpu.sync_copy(den_r.at[pl.ds(0, NT)], inv_loc)
    pltpu.sync_copy(den_r.at[pl.ds(NT, NT)], q_loc)

    zv = jnp.zeros((16,), jnp.float32)

    def inv_body(k, _):
        s = pl.ds(k * 16, 16)
        inv_loc[s] = 1.0 / (inv_loc[s] + q_loc[s] + 1e-16)
        q_loc[s] = zv
        return 0

    lax.fori_loop(0, NT // 16, inv_body, 0, unroll=8)

    def batch_body(b, _):
        row0 = base_row + b * (CB // C)
        pltpu.sync_copy(src_r.at[pl.ds(row0, CB // C)], sbuf_src)
        pltpu.sync_copy(dst_r.at[pl.ds(row0, CB // C)], sbuf_dst)
        eoff = row0 * C
        for h in range(H):
            pltpu.sync_copy(ex_r.at[pl.ds(h * EP + eoff, CB)], exb.at[h])

        def grp_body(g, _):
            r = g // (C // 16)
            l0 = (g % (C // 16)) * 16
            srcv = sbuf_src[r, pl.ds(l0, 16)]
            dstv = sbuf_dst[r, pl.ds(l0, 16)]
            for h in range(H):
                exv = exb[h, pl.ds(g * 16, 16)]
                di = dstv + h * NP
                pi = srcv + h * NP
                invv = plsc.load_gather(inv_loc, [di])
                pv = plsc.load_gather(p_loc, [pi])
                plsc.addupdate_scatter(q_loc, [di], exv * invv * pv)
            return 0

        lax.fori_loop(0, CB // 16, grp_body, 0)
        return 0

    lax.fori_loop(0, NB, batch_body, 0)

    plsc.subcore_barrier()
    _tree_reduce(q_loc, tmpa, tmpb0, tmpb1, shared_all, q_r, cid, sid, semr)


def _pass2(src2, dst2, ex, den, ptflat):
    k = pl.kernel(
        _pass2_body,
        out_type=[jax.ShapeDtypeStruct((2 * NT,), jnp.float32)],
        mesh=plsc.VectorSubcoreMesh(core_axis_name="c", subcore_axis_name="s"),
        compiler_params=pltpu.CompilerParams(use_tc_tiling_on_sc=False, needs_layout_passes=False),
        scratch_types=[
            pltpu.VMEM((CB // C, C), jnp.int32),
            pltpu.VMEM((CB // C, C), jnp.int32),
            pltpu.VMEM((H, CB), jnp.float32),
            pltpu.VMEM((NT,), jnp.float32),
            pltpu.VMEM((NT,), jnp.float32),
            pltpu.VMEM((NT,), jnp.float32),
            pltpu.VMEM((PSL,), jnp.float32),
            pltpu.VMEM((PSL,), jnp.float32),
            pltpu.VMEM((PSL,), jnp.float32),
            pltpu.VMEM_SHARED((16, PLEN), jnp.float32),
            pltpu.SemaphoreType.DMA,
        ],
    )
    return k(src2, dst2, ex, den, ptflat)


def _final_body(q_ref, out_ref):
    q = q_ref[...]
    qs = q[0:240] + q[240:480]
    s = (qs[0:80] + qs[80:160] + qs[160:240]) * (1.0 / 3.0)
    n = (lax.broadcasted_iota(jnp.int32, (80, 128), 0) * 128
         + lax.broadcasted_iota(jnp.int32, (80, 128), 1))
    s = jnp.where(n < N, s, -jnp.inf)
    m = jnp.max(s)
    e = jnp.exp(s - m)
    out_ref[...] = e / jnp.sum(e)


def _final(q):
    return pl.pallas_call(
        _final_body,
        out_shape=jax.ShapeDtypeStruct((80, 128), jnp.float32),
    )(q.reshape(480, 128))


def kernel(edge_index, edge_attr, path_emb, sim_w, Wl, bl, Wr, br, We, att,
           bias_gnn, W_out, b_out):
    f32 = jnp.float32
    src = edge_index[0]
    dst = edge_index[1]
    f = edge_attr[:, 0] * jax.nn.softmax(sim_w)[0]

    pad = EP - E
    src2 = jnp.concatenate([src, jnp.zeros((pad,), src.dtype)]).reshape(EP // C, C)
    dst2 = jnp.concatenate([dst, jnp.full((pad,), N, dst.dtype)]).reshape(EP // C, C)
    f2 = jnp.concatenate([f, jnp.zeros((pad,), f32)]).reshape(EP // C, C)

    pe_p = jnp.pad(path_emb, ((0, NP - N), (0, 0)))
    w3t = jnp.kron(jnp.eye(H, dtype=f32), W_out[:, 0][None, :])

    XL, XR, PT = _prep(pe_p, Wl, Wr, bl[None, :], br[None, :], w3t)

    ex, den = _pass1(src2, dst2, f2, XL, XR,
                     We.reshape(-1).astype(jnp.bfloat16),
                     att.reshape(-1).astype(jnp.bfloat16))
    (q,) = _pass2(src2, dst2, ex, den, PT)
    w = _final(q)
    return w.reshape(-1)[:N]


# speedup vs baseline: 1.0554x; 1.0554x over previous
import functools

import jax
import jax.numpy as jnp
from jax import lax
from jax.experimental import pallas as pl
from jax.experimental.pallas import tpu as pltpu
from jax.experimental.pallas import tpu_sc as plsc

N = 10000
NP = 10240
E = 160000
EP = 163840
H = 3
HID = 256
D = 768
NW = 32
TPW = EP // NW
NB = 5
CB = TPW // NB
C = 32
KPB = CB // C
NT = H * NP
SLICE = NT // 16
PARTS = 4
PLEN = NT // PARTS
PSL = PLEN // 16


def _prep_body(pe_ref, wl_ref, wr_ref, bl_ref, br_ref, w3t_ref, xl_out, xr_out, pt_out):
    hi = jax.lax.Precision.HIGHEST
    xl = jnp.dot(pe_ref[...], wl_ref[...], preferred_element_type=jnp.float32) \
        + bl_ref[...]
    xl_out[...] = xl.astype(jnp.bfloat16)
    xr = jnp.dot(pe_ref[...], wr_ref[...], preferred_element_type=jnp.float32) \
        + br_ref[...]
    xr_out[...] = xr.astype(jnp.bfloat16)
    pt_out[...] = jax.lax.dot_general(
        w3t_ref[...], xl, (((1,), (1,)), ((), ())),
        preferred_element_type=jnp.float32, precision=hi)


def _prep(pe_p, Wl, Wr, bl2, br2, w3t):
    bm = 256
    grid = (NP // bm,)
    return pl.pallas_call(
        _prep_body,
        grid=grid,
        in_specs=[
            pl.BlockSpec((bm, 256), lambda i: (i, 0)),
            pl.BlockSpec((256, D), lambda i: (0, 0)),
            pl.BlockSpec((256, D), lambda i: (0, 0)),
            pl.BlockSpec((1, D), lambda i: (0, 0)),
            pl.BlockSpec((1, D), lambda i: (0, 0)),
            pl.BlockSpec((H, D), lambda i: (0, 0)),
        ],
        out_specs=[
            pl.BlockSpec((bm, D), lambda i: (i, 0)),
            pl.BlockSpec((bm, D), lambda i: (i, 0)),
            pl.BlockSpec((H, bm), lambda i: (0, i)),
        ],
        out_shape=[
            jax.ShapeDtypeStruct((NP, D), jnp.bfloat16),
            jax.ShapeDtypeStruct((NP, D), jnp.bfloat16),
            jax.ShapeDtypeStruct((H, NP), jnp.float32),
        ],
    )(pe_p, Wl, Wr, bl2, br2, w3t)


def _worker_id():
    return lax.axis_index("s") * 2 + lax.axis_index("c")


def _tree_reduce(loc, tmpa, tmpb0, tmpb1, shared_all, out_ref, cid, sid, semr):
    sl0 = sid * PSL

    def fetch(t, buf):
        pltpu.async_copy(shared_all.at[t, pl.ds(sl0, PSL)], buf, semr)

    def wait(buf):
        pltpu.make_async_copy(shared_all.at[0, pl.ds(sl0, PSL)], buf, semr).wait()

    def addin(buf):
        def add_body(k, _):
            s = pl.ds(k * 16, 16)
            tmpa[s] = tmpa[s] + buf[s]
            return 0

        lax.fori_loop(0, PSL // 16, add_body, 0, unroll=8)

    def part_body(part, _):
        pltpu.sync_copy(loc.at[pl.ds(part * PLEN, PLEN)], shared_all.at[sid])
        plsc.subcore_barrier()
        pltpu.sync_copy(shared_all.at[0, pl.ds(sl0, PSL)], tmpa)
        fetch(1, tmpb0)
        for pi in range(7):
            wait(tmpb0)
            fetch(2 * pi + 2, tmpb1)
            addin(tmpb0)
            wait(tmpb1)
            fetch(2 * pi + 3, tmpb0)
            addin(tmpb1)
        wait(tmpb0)
        addin(tmpb0)
        pltpu.sync_copy(tmpa, out_ref.at[pl.ds(cid * NT + part * PLEN + sl0, PSL)])
        plsc.subcore_barrier()
        return 0

    lax.fori_loop(0, PARTS, part_body, 0)


def _pass1_body(src_r, dst_r, f_r, xl_r, xr_r, we_r, att_r,
                ex_r, den_r,
                sbuf_src, sbuf_dst, sbuf_f, rows_la, rows_ra, rows_lb, rows_rb,
                webuf, attbuf, exbig, den_loc, tmpa, tmpb0, tmpb1, shared_all,
                sem_la, sem_ra, sem_lb, sem_rb):
    cid = lax.axis_index("c")
    sid = lax.axis_index("s")
    wid = _worker_id()
    base_row = wid * (TPW // C)

    pltpu.sync_copy(we_r, webuf)
    pltpu.sync_copy(att_r, attbuf)

    zv = jnp.zeros((16,), jnp.float32)

    def zero_body(k, _):
        den_loc[pl.ds(k * 16, 16)] = zv
        return 0

    lax.fori_loop(0, NT // 16, zero_body, 0, unroll=8)

    lane16 = lax.iota(jnp.int32, 16)
    perms = [(lane16 + s) & 15 for s in (8, 4, 2, 1)]

    def start(k, buf_l, buf_r, sl, sr):
        pltpu.async_copy(xl_r.at[sbuf_src.at[k]], buf_l, sl)
        pltpu.async_copy(xr_r.at[sbuf_dst.at[k]], buf_r, sr)

    def drain(buf_l, buf_r, sl, sr):
        pltpu.make_async_copy(xl_r.at[pl.ds(0, C)], buf_l, sl).wait()
        pltpu.make_async_copy(xr_r.at[pl.ds(0, C)], buf_r, sr).wait()

    p2v = jnp.full((32,), 0.2, jnp.bfloat16)

    def compute(k, rows_l, rows_r):
        for j in range(C // 16):
            fv = sbuf_f[k, pl.ds(j * 16, 16)]
            dstv = sbuf_dst[k, pl.ds(j * 16, 16)]
            for h in range(H):
                we_regs = [webuf[pl.ds(h * HID + c * 32, 32)] for c in range(8)]
                att_regs = [attbuf[pl.ds(h * HID + c * 32, 32)] for c in range(8)]

                def e_body(e, alpha_vec):
                    esp = jnp.full((16,), e, jnp.int32)
                    fsp = fv.at[esp].get(mode="promise_in_bounds")
                    fsb = plsc.pack(fsp, fsp, format=plsc.PackFormat.INTERLEAVED)
                    acc_bf = jnp.zeros((32,), jnp.bfloat16)
                    for c in range(8):
                        s = pl.ds(h * HID + c * 32, 32)
                        y = (rows_l[j * 16 + e, s] + rows_r[j * 16 + e, s]
                             + fsb * we_regs[c])
                        y = jnp.maximum(y, p2v * y)
                        acc_bf = acc_bf + att_regs[c] * y
                    lo, hi = plsc.unpack(acc_bf, format=plsc.PackFormat.INTERLEAVED)
                    acc = lo + hi
                    for p in perms:
                        acc = acc + acc.at[p].get(mode="promise_in_bounds")
                    return jnp.where(lane16 == e, acc, alpha_vec)

                alpha = lax.fori_loop(0, 16, e_body, jnp.zeros((16,), jnp.float32),
                                      unroll=2)
                exv = jnp.exp(alpha)
                exbig[h, pl.ds((k % KPB) * C + j * 16, 16)] = exv
                plsc.addupdate_scatter(den_loc, [dstv + h * NP], exv)

    def batch_body(b, _):
        row0 = base_row + b * KPB
        pltpu.sync_copy(src_r.at[pl.ds(row0, KPB)], sbuf_src)
        pltpu.sync_copy(dst_r.at[pl.ds(row0, KPB)], sbuf_dst)
        pltpu.sync_copy(f_r.at[pl.ds(row0, KPB)], sbuf_f)

        start(0, rows_la, rows_ra, sem_la, sem_ra)

        def pair_body(p, _):
            k0 = 2 * p
            drain(rows_la, rows_ra, sem_la, sem_ra)
            start(k0 + 1, rows_lb, rows_rb, sem_lb, sem_rb)
            compute(k0, rows_la, rows_ra)
            drain(rows_lb, rows_rb, sem_lb, sem_rb)
            start(jnp.minimum(k0 + 2, KPB - 1), rows_la, rows_ra, sem_la, sem_ra)
            compute(k0 + 1, rows_lb, rows_rb)
            return 0

        lax.fori_loop(0, KPB // 2, pair_body, 0)
        drain(rows_la, rows_ra, sem_la, sem_ra)
        for h in range(H):
            pltpu.sync_copy(exbig.at[h], ex_r.at[pl.ds(h * EP + row0 * C, CB)])
        return 0

    lax.fori_loop(0, NB, batch_body, 0)

    plsc.subcore_barrier()
    _tree_reduce(den_loc, tmpa, tmpb0, tmpb1, shared_all, den_r, cid, sid, sem_la)


def _pass1(src2, dst2, f2, XL, XR, wef, attf):
    k = pl.kernel(
        _pass1_body,
        out_type=[
            jax.ShapeDtypeStruct((H * EP,), jnp.float32),
            jax.ShapeDtypeStruct((2 * NT,), jnp.float32),
        ],
        mesh=plsc.VectorSubcoreMesh(core_axis_name="c", subcore_axis_name="s"),
        compiler_params=pltpu.CompilerParams(use_tc_tiling_on_sc=False, needs_layout_passes=False),
        scratch_types=[
            pltpu.VMEM((KPB, C), jnp.int32),
            pltpu.VMEM((KPB, C), jnp.int32),
            pltpu.VMEM((KPB, C), jnp.float32),
            pltpu.VMEM((C, D), jnp.bfloat16),
            pltpu.VMEM((C, D), jnp.bfloat16),
            pltpu.VMEM((C, D), jnp.bfloat16),
            pltpu.VMEM((C, D), jnp.bfloat16),
            pltpu.VMEM((D,), jnp.bfloat16),
            pltpu.VMEM((D,), jnp.bfloat16),
            pltpu.VMEM((H, CB), jnp.float32),
            pltpu.VMEM((NT,), jnp.float32),
            pltpu.VMEM((PSL,), jnp.float32),
            pltpu.VMEM((PSL,), jnp.float32),
            pltpu.VMEM((PSL,), jnp.float32),
            pltpu.VMEM_SHARED((16, PLEN), jnp.float32),
            pltpu.SemaphoreType.DMA,
            pltpu.SemaphoreType.DMA,
            pltpu.SemaphoreType.DMA,
            pltpu.SemaphoreType.DMA,
        ],
    )
    return k(src2, dst2, f2, XL, XR, wef, attf)


def _pass2_body(src_r, dst_r, ex_r, den_r, p_r,
                q_r,
                sbuf_src, sbuf_dst, exb, p_loc, inv_loc, q_loc,
                tmpa, tmpb0, tmpb1, shared_all, semr):
    cid = lax.axis_index("c")
    sid = lax.axis_index("s")
    wid = _worker_id()
    base_row = wid * (TPW // C)

    for h in range(H):
        pltpu.sync_copy(p_r.at[h], p_loc.at[pl.ds(h * NP, NP)])
    pltpu.sync_copy(den_r.at[pl.ds(0, NT)], inv_loc)
    pltpu.sync_copy(den_r.at[pl.ds(NT, NT)], q_loc)

    zv = jnp.zeros((16,), jnp.float32)

    def inv_body(k, _):
        s = pl.ds(k * 16, 16)
        inv_loc[s] = 1.0 / (inv_loc[s] + q_loc[s] + 1e-16)
        q_loc[s] = zv
        return 0

    lax.fori_loop(0, NT // 16, inv_body, 0, unroll=8)

    def batch_body(b, _):
        row0 = base_row + b * (CB // C)
        pltpu.sync_copy(src_r.at[pl.ds(row0, CB // C)], sbuf_src)
        pltpu.sync_copy(dst_r.at[pl.ds(row0, CB // C)], sbuf_dst)
        eoff = row0 * C
        for h in range(H):
            pltpu.sync_copy(ex_r.at[pl.ds(h * EP + eoff, CB)], exb.at[h])

        def grp_body(g, _):
            r = g // (C // 16)
            l0 = (g % (C // 16)) * 16
            srcv = sbuf_src[r, pl.ds(l0, 16)]
            dstv = sbuf_dst[r, pl.ds(l0, 16)]
            for h in range(H):
                exv = exb[h, pl.ds(g * 16, 16)]
                di = dstv + h * NP
                pi = srcv + h * NP
                invv = plsc.load_gather(inv_loc, [di])
                pv = plsc.load_gather(p_loc, [pi])
                plsc.addupdate_scatter(q_loc, [di], exv * invv * pv)
            return 0

        lax.fori_loop(0, CB // 16, grp_body, 0)
        return 0

    lax.fori_loop(0, NB, batch_body, 0)

    plsc.subcore_barrier()
    _tree_reduce(q_loc, tmpa, tmpb0, tmpb1, shared_all, q_r, cid, sid, semr)


def _pass2(src2, dst2, ex, den, ptflat):
    k = pl.kernel(
        _pass2_body,
        out_type=[jax.ShapeDtypeStruct((2 * NT,), jnp.float32)],
        mesh=plsc.VectorSubcoreMesh(core_axis_name="c", subcore_axis_name="s"),
        compiler_params=pltpu.CompilerParams(use_tc_tiling_on_sc=False, needs_layout_passes=False),
        scratch_types=[
            pltpu.VMEM((CB // C, C), jnp.int32),
            pltpu.VMEM((CB // C, C), jnp.int32),
            pltpu.VMEM((H, CB), jnp.float32),
            pltpu.VMEM((NT,), jnp.float32),
            pltpu.VMEM((NT,), jnp.float32),
            pltpu.VMEM((NT,), jnp.float32),
            pltpu.VMEM((PSL,), jnp.float32),
            pltpu.VMEM((PSL,), jnp.float32),
            pltpu.VMEM((PSL,), jnp.float32),
            pltpu.VMEM_SHARED((16, PLEN), jnp.float32),
            pltpu.SemaphoreType.DMA,
        ],
    )
    return k(src2, dst2, ex, den, ptflat)


def _final_body(q_ref, out_ref):
    q = q_ref[...]
    qs = q[0:240] + q[240:480]
    s = (qs[0:80] + qs[80:160] + qs[160:240]) * (1.0 / 3.0)
    n = (lax.broadcasted_iota(jnp.int32, (80, 128), 0) * 128
         + lax.broadcasted_iota(jnp.int32, (80, 128), 1))
    s = jnp.where(n < N, s, -jnp.inf)
    m = jnp.max(s)
    e = jnp.exp(s - m)
    out_ref[...] = e / jnp.sum(e)


def _final(q):
    return pl.pallas_call(
        _final_body,
        out_shape=jax.ShapeDtypeStruct((80, 128), jnp.float32),
    )(q.reshape(480, 128))


def kernel(edge_index, edge_attr, path_emb, sim_w, Wl, bl, Wr, br, We, att,
           bias_gnn, W_out, b_out):
    f32 = jnp.float32
    src = edge_index[0]
    dst = edge_index[1]
    f = edge_attr[:, 0] * jax.nn.softmax(sim_w)[0]

    pad = EP - E
    src2 = jnp.concatenate([src, jnp.zeros((pad,), src.dtype)]).reshape(EP // C, C)
    dst2 = jnp.concatenate([dst, jnp.full((pad,), N, dst.dtype)]).reshape(EP // C, C)
    f2 = jnp.concatenate([f, jnp.zeros((pad,), f32)]).reshape(EP // C, C)

    pe_p = jnp.pad(path_emb, ((0, NP - N), (0, 0)))
    w3t = jnp.kron(jnp.eye(H, dtype=f32), W_out[:, 0][None, :])

    XL, XR, PT = _prep(pe_p, Wl, Wr, bl[None, :], br[None, :], w3t)

    ex, den = _pass1(src2, dst2, f2, XL, XR,
                     We.reshape(-1).astype(jnp.bfloat16),
                     att.reshape(-1).astype(jnp.bfloat16))
    (q,) = _pass2(src2, dst2, ex, den, PT)
    w = _final(q)
    return w.reshape(-1)[:N]
